# baseline (speedup 1.0000x reference)
import dataclasses
import functools

import jax
import jax.numpy as jnp
from jax import lax
from jax.experimental import pallas as pl
from jax.experimental.pallas import tpu as pltpu
from jax.experimental.pallas import tpu_sc as plsc

N = 10000
E = 320000
D = 128
VOCAB = 20215
G = 512

NPAD = 10240
EPAD = 327680
TRASH = N
GPAD = 640
NW = 32
C0 = 96
C1 = 224

_mesh = plsc.VectorSubcoreMesh(core_axis_name="c", subcore_axis_name="s")

_sc_params = pltpu.CompilerParams()
if "needs_layout_passes" in pltpu.CompilerParams.__dataclass_fields__:
    _sc_params = dataclasses.replace(_sc_params, needs_layout_passes=False)

_f32 = jnp.float32
_i32 = jnp.int32


@functools.partial(
    pl.kernel,
    out_type=[
        jax.ShapeDtypeStruct((NPAD, D), _f32),
        jax.ShapeDtypeStruct((2, NPAD, D), _f32),
        jax.ShapeDtypeStruct((NW * NPAD,), _f32),
        jax.ShapeDtypeStruct((NW * GPAD,), _f32),
    ],
    mesh=_mesh,
    compiler_params=_sc_params,
    scratch_types=[
        pltpu.VMEM_SHARED((NPAD, D), _f32),
        pltpu.VMEM((NPAD,), _i32),
        pltpu.VMEM((NPAD,), _f32),
        pltpu.VMEM((GPAD,), _f32),
        pltpu.VMEM((128,), _i32),
        pltpu.VMEM((128,), _i32),
        pltpu.VMEM((64,), _i32),
        pltpu.VMEM((64,), _i32),
        pltpu.VMEM((64,), _i32),
        pltpu.VMEM((64,), _i32),
        pltpu.VMEM((64,), _i32),
        pltpu.VMEM((64, D), _f32),
        pltpu.VMEM((64, D), _f32),
        pltpu.SemaphoreType.DMA,
        pltpu.SemaphoreType.DMA,
    ],
)
def _agg_kernel(emb_hbm, x_hbm, sd_hbm, batch_hbm,
                h_hbm, p_hbm, cnt_hbm, gc_hbm,
                agg_sh, x_v, cnt_v, gcv_v, sd_a, sd_b, dst_a, dst_b,
                eidx_a, eidx_b, bidx_v, msg_a, msg_b, sem_a, sem_b):
    c = lax.axis_index("c")
    s = lax.axis_index("s")
    wid = c * 16 + s

    @pl.loop(0, 64)
    def _(r):
        @pl.loop(0, D, step=16)
        def _(cc):
            msg_a[r, pl.ds(cc, 16)] = jnp.zeros((16,), _f32)

    rb = pl.multiple_of(s * 640, 64)

    @pl.loop(0, 640, step=64)
    def _(r):
        pltpu.sync_copy(msg_a, agg_sh.at[pl.ds(rb + r, 64)])

    @pl.loop(0, NPAD, step=16)
    def _(r):
        cnt_v[pl.ds(r, 16)] = jnp.zeros((16,), _f32)

    @pl.loop(0, GPAD, step=16)
    def _(r):
        gcv_v[pl.ds(r, 16)] = jnp.zeros((16,), _f32)

    plsc.subcore_barrier()

    pltpu.sync_copy(x_hbm, x_v)

    nbase = wid * 320

    @pl.loop(0, 5)
    def _(k):
        b = pl.multiple_of(nbase + k * 64, 64)
        ga = pltpu.async_copy(emb_hbm.at[x_v.at[pl.ds(b, 64)]], msg_a, sem_a)
        pltpu.sync_copy(batch_hbm.at[pl.ds(b, 64)], bidx_v)

        @pl.loop(0, 64, step=16)
        def _(j):
            plsc.addupdate_scatter(gcv_v, [bidx_v[pl.ds(j, 16)]],
                                   jnp.ones((16,), _f32))

        ga.wait()
        pltpu.sync_copy(msg_a, h_hbm.at[pl.ds(b, 64)])

    base_chunk = jnp.where(c == 0, s * C0, 16 * C0 + s * C1)
    nchunks = jnp.where(c == 0, C0, C1)
    ebase = base_chunk * 128

    @pl.loop(0, nchunks, step=2)
    def _(k):
        bA = pl.multiple_of(ebase + k * 128, 128)
        pltpu.sync_copy(sd_hbm.at[pl.ds(bA, 128)], sd_a)

        @pl.loop(0, 64, step=16)
        def _(j):
            eidx_a[pl.ds(j, 16)] = plsc.load_gather(x_v, [sd_a[pl.ds(j, 16)]])
            dv = sd_a[pl.ds(64 + j, 16)]
            dst_a[pl.ds(j, 16)] = dv
            plsc.addupdate_scatter(cnt_v, [dv], jnp.ones((16,), _f32))

        ga = pltpu.async_copy(emb_hbm.at[eidx_a], msg_a, sem_a)
        bB = pl.multiple_of(bA + 128, 128)
        pltpu.sync_copy(sd_hbm.at[pl.ds(bB, 128)], sd_b)

        @pl.loop(0, 64, step=16)
        def _(j):
            eidx_b[pl.ds(j, 16)] = plsc.load_gather(x_v, [sd_b[pl.ds(j, 16)]])
            dv = sd_b[pl.ds(64 + j, 16)]
            dst_b[pl.ds(j, 16)] = dv
            plsc.addupdate_scatter(cnt_v, [dv], jnp.ones((16,), _f32))

        gb = pltpu.async_copy(emb_hbm.at[eidx_b], msg_b, sem_b)
        ga.wait()
        pltpu.sync_copy(msg_a, agg_sh.at[dst_a], add=True)
        gb.wait()
        pltpu.sync_copy(msg_b, agg_sh.at[dst_b], add=True)

    plsc.subcore_barrier()

    cb = pl.multiple_of(wid * NPAD, 128)
    pltpu.sync_copy(cnt_v, cnt_hbm.at[pl.ds(cb, NPAD)])
    gcb = pl.multiple_of(wid * GPAD, 128)
    pltpu.sync_copy(gcv_v, gc_hbm.at[pl.ds(gcb, GPAD)])
    pltpu.sync_copy(agg_sh.at[pl.ds(rb, 640)], p_hbm.at[c, pl.ds(rb, 640)])


def _tc_body(p, cnt, h, wlT, wrT, bl, o):
    deg = jnp.sum(cnt[...], axis=0).reshape(_BR, 1)
    deg = jnp.maximum(deg, 1.0)
    mean = (p[0] + p[1]) / deg
    acc = jnp.dot(mean, wlT[...], preferred_element_type=_f32)
    acc = acc + jnp.dot(h[...], wrT[...], preferred_element_type=_f32)
    o[...] = jnp.maximum(acc + bl[0:1, :], 0.0)


_BR = 512

_tc_call = pl.pallas_call(
    _tc_body,
    grid=(NPAD // _BR,),
    in_specs=[
        pl.BlockSpec((2, _BR, D), lambda i: (0, i, 0)),
        pl.BlockSpec((NW, _BR), lambda i: (0, i)),
        pl.BlockSpec((_BR, D), lambda i: (i, 0)),
        pl.BlockSpec((D, D), lambda i: (0, 0)),
        pl.BlockSpec((D, D), lambda i: (0, 0)),
        pl.BlockSpec((8, D), lambda i: (0, 0)),
    ],
    out_specs=pl.BlockSpec((_BR, D), lambda i: (i, 0)),
    out_shape=jax.ShapeDtypeStruct((NPAD, D), _f32),
)


@functools.partial(
    pl.kernel,
    out_type=jax.ShapeDtypeStruct((G,), _f32),
    mesh=_mesh,
    compiler_params=_sc_params,
    scratch_types=[
        pltpu.VMEM((NW * GPAD,), _f32),
        pltpu.VMEM((GPAD + 16,), _f32),
        pltpu.VMEM((NPAD,), _i32),
        pltpu.VMEM((64, D), _f32),
        pltpu.VMEM((16, D), _f32),
        pltpu.VMEM((16, D), _f32),
        pltpu.VMEM((2 * D,), _f32),
        pltpu.VMEM((16,), _f32),
        pltpu.VMEM((16,), _f32),
        pltpu.SemaphoreType.DMA,
    ],
)
def _pool_kernel(out_hbm, batch_hbm, gc_all_hbm, wout_hbm, bout_hbm,
                 res_hbm, gc_v, gct_v, batch_v, rowbuf, accmax, accsum,
                 wout_v, bout_v, res_v, sem):
    c = lax.axis_index("c")
    s = lax.axis_index("s")
    wid = c * 16 + s
    wid16 = wid * 16

    pltpu.sync_copy(gc_all_hbm, gc_v)

    @pl.loop(0, GPAD, step=16)
    def _(r):
        t = jnp.zeros((16,), _f32)
        for w in range(NW):
            t = t + gc_v[pl.ds(w * GPAD + r, 16)]
        gct_v[pl.ds(r, 16)] = t

    gct_v[pl.ds(GPAD, 16)] = jnp.zeros((16,), _f32)
    pltpu.sync_copy(batch_hbm, batch_v)
    pltpu.sync_copy(wout_hbm, wout_v)
    pltpu.sync_copy(bout_hbm, bout_v)

    @pl.loop(0, 16)
    def _(r):
        @pl.loop(0, D, step=16)
        def _(cc):
            accmax[r, pl.ds(cc, 16)] = jnp.zeros((16,), _f32)
            accsum[r, pl.ds(cc, 16)] = jnp.zeros((16,), _f32)

    def _cnt_at(g):
        return gct_v[pl.ds(g, 16)][0]

    start_f = lax.fori_loop(0, wid16, lambda g, a: a + _cnt_at(g), 0.0)
    cgs = [_cnt_at(wid16 + gl) for gl in range(16)]
    total_f = cgs[0]
    for gl in range(1, 16):
        total_f = total_f + cgs[gl]
    start = start_f.astype(_i32)
    total = total_f.astype(_i32)

    end = start + total
    w0 = start // 64

    def _chunk(i, carry):
        p = pl.multiple_of((w0 + i) * 64, 64)
        pltpu.sync_copy(out_hbm.at[pl.ds(p, 64)], rowbuf)
        lo = jnp.maximum(start - p, 0)
        hi = jnp.minimum(end - p, 64)

        @pl.loop(0, 64)
        def _(r):
            @pl.when((r >= lo) & (r < hi))
            def _():
                gloc = batch_v[pl.ds(p + r, 16)][0] - wid16

                @pl.loop(0, D, step=16)
                def _(cc):
                    v = rowbuf[r, pl.ds(cc, 16)]
                    accmax[gloc, pl.ds(cc, 16)] = jnp.maximum(
                        accmax[gloc, pl.ds(cc, 16)], v)
                    accsum[gloc, pl.ds(cc, 16)] = accsum[gloc, pl.ds(cc, 16)] + v

        return carry

    nch = jnp.where(total > 0, (end + 63) // 64 - w0, 0)
    lax.fori_loop(0, nch, _chunk, 0)

    lane = lax.iota(_i32, 16)
    bias = bout_v[...][0]
    logits = jnp.zeros((16,), _f32)
    for gl in range(16):
        denom = jnp.maximum(cgs[gl], 1.0)
        acc = jnp.zeros((16,), _f32)
        for cc in range(8):
            acc = acc + accmax[gl, pl.ds(cc * 16, 16)] * wout_v[pl.ds(cc * 16, 16)]
            acc = acc + accsum[gl, pl.ds(cc * 16, 16)] * (
                wout_v[pl.ds(D + cc * 16, 16)] / denom)
        t = jnp.sum(acc) + bias
        logits = jnp.where(lane == gl, t, logits)

    res_v[...] = 1.0 / (1.0 + jnp.exp(-logits))
    pltpu.sync_copy(res_v, res_hbm.at[pl.ds(wid16, 16)])


def kernel(x, edge_index, batch, emb_table, W_l, b_l, W_r, W_out, b_out):
    xp = jnp.concatenate([x[:, 0], jnp.zeros((NPAD - N,), _i32)])
    srcp = jnp.concatenate([edge_index[0], jnp.zeros((EPAD - E,), _i32)])
    dstp = jnp.concatenate([edge_index[1], jnp.full((EPAD - E,), TRASH, _i32)])
    sdp = jnp.stack([srcp.reshape(-1, 64), dstp.reshape(-1, 64)],
                    axis=1).reshape(-1)
    batchp = jnp.concatenate([batch, jnp.full((NPAD - N,), G, _i32)])
    wlT = W_l.T
    wrT = W_r.T
    bl8 = jnp.broadcast_to(b_l[None, :], (8, D))
    woutv = W_out.reshape(-1)
    boutv = jnp.pad(b_out, (0, 15))

    h, p, cnt, gc = _agg_kernel(emb_table, xp, sdp, batchp)
    cnt2 = cnt.reshape(NW, NPAD)
    out = _tc_call(p, cnt2, h, wlT, wrT, bl8)
    res = _pool_kernel(out, batchp, gc.reshape(-1), woutv, boutv)
    return res


# speedup vs baseline: 1.2232x; 1.2232x over previous
import dataclasses
import functools

import jax
import jax.numpy as jnp
from jax import lax
from jax.experimental import pallas as pl
from jax.experimental.pallas import tpu as pltpu
from jax.experimental.pallas import tpu_sc as plsc

N = 10000
E = 320000
D = 128
VOCAB = 20215
G = 512

NPAD = 10240
EPAD = 327680
TRASH = N
GPAD = 640
NW = 32
C0 = 224
C1 = 96

_mesh = plsc.VectorSubcoreMesh(core_axis_name="c", subcore_axis_name="s")

_sc_params = pltpu.CompilerParams()
if "needs_layout_passes" in pltpu.CompilerParams.__dataclass_fields__:
    _sc_params = dataclasses.replace(_sc_params, needs_layout_passes=False)

_f32 = jnp.float32
_i32 = jnp.int32


@functools.partial(
    pl.kernel,
    out_type=[
        jax.ShapeDtypeStruct((NPAD, D), _f32),
        jax.ShapeDtypeStruct((2, NPAD, D), _f32),
        jax.ShapeDtypeStruct((NW * NPAD,), _f32),
        jax.ShapeDtypeStruct((NW * GPAD,), _f32),
    ],
    mesh=_mesh,
    compiler_params=_sc_params,
    scratch_types=[
        pltpu.VMEM_SHARED((NPAD, D), _f32),
        pltpu.VMEM((NPAD,), _i32),
        pltpu.VMEM((NPAD,), _f32),
        pltpu.VMEM((GPAD,), _f32),
        pltpu.VMEM((128,), _i32),
        pltpu.VMEM((128,), _i32),
        pltpu.VMEM((64,), _i32),
        pltpu.VMEM((64,), _i32),
        pltpu.VMEM((64,), _i32),
        pltpu.VMEM((64,), _i32),
        pltpu.VMEM((64,), _i32),
        pltpu.VMEM((64, D), _f32),
        pltpu.VMEM((64, D), _f32),
        pltpu.SemaphoreType.DMA,
        pltpu.SemaphoreType.DMA,
    ],
)
def _agg_kernel(emb_hbm, x_hbm, sd_hbm, batch_hbm,
                h_hbm, p_hbm, cnt_hbm, gc_hbm,
                agg_sh, x_v, cnt_v, gcv_v, sd_a, sd_b, dst_a, dst_b,
                eidx_a, eidx_b, bidx_v, msg_a, msg_b, sem_a, sem_b):
    c = lax.axis_index("c")
    s = lax.axis_index("s")
    wid = c * 16 + s

    @pl.loop(0, 64)
    def _(r):
        @pl.loop(0, D, step=16)
        def _(cc):
            msg_a[r, pl.ds(cc, 16)] = jnp.zeros((16,), _f32)

    rb = pl.multiple_of(s * 640, 64)

    @pl.loop(0, 640, step=64)
    def _(r):
        pltpu.sync_copy(msg_a, agg_sh.at[pl.ds(rb + r, 64)])

    @pl.loop(0, NPAD, step=16)
    def _(r):
        cnt_v[pl.ds(r, 16)] = jnp.zeros((16,), _f32)

    @pl.loop(0, GPAD, step=16)
    def _(r):
        gcv_v[pl.ds(r, 16)] = jnp.zeros((16,), _f32)

    plsc.subcore_barrier()

    pltpu.sync_copy(x_hbm, x_v)

    nbase = wid * 320

    @pl.loop(0, 5)
    def _(k):
        b = pl.multiple_of(nbase + k * 64, 64)
        ga = pltpu.async_copy(emb_hbm.at[x_v.at[pl.ds(b, 64)]], msg_a, sem_a)
        pltpu.sync_copy(batch_hbm.at[pl.ds(b, 64)], bidx_v)

        @pl.loop(0, 64, step=16)
        def _(j):
            plsc.addupdate_scatter(gcv_v, [bidx_v[pl.ds(j, 16)]],
                                   jnp.ones((16,), _f32))

        ga.wait()
        pltpu.sync_copy(msg_a, h_hbm.at[pl.ds(b, 64)])

    base_chunk = jnp.where(c == 0, s * C0, 16 * C0 + s * C1)
    nchunks = jnp.where(c == 0, C0, C1)
    ebase = base_chunk * 128

    @pl.loop(0, nchunks, step=2)
    def _(k):
        bA = pl.multiple_of(ebase + k * 128, 128)
        pltpu.sync_copy(sd_hbm.at[pl.ds(bA, 128)], sd_a)

        @pl.loop(0, 64, step=16)
        def _(j):
            eidx_a[pl.ds(j, 16)] = plsc.load_gather(x_v, [sd_a[pl.ds(j, 16)]])
            dv = sd_a[pl.ds(64 + j, 16)]
            dst_a[pl.ds(j, 16)] = dv
            plsc.addupdate_scatter(cnt_v, [dv], jnp.ones((16,), _f32))

        ga = pltpu.async_copy(emb_hbm.at[eidx_a], msg_a, sem_a)
        bB = pl.multiple_of(bA + 128, 128)
        pltpu.sync_copy(sd_hbm.at[pl.ds(bB, 128)], sd_b)

        @pl.loop(0, 64, step=16)
        def _(j):
            eidx_b[pl.ds(j, 16)] = plsc.load_gather(x_v, [sd_b[pl.ds(j, 16)]])
            dv = sd_b[pl.ds(64 + j, 16)]
            dst_b[pl.ds(j, 16)] = dv
            plsc.addupdate_scatter(cnt_v, [dv], jnp.ones((16,), _f32))

        gb = pltpu.async_copy(emb_hbm.at[eidx_b], msg_b, sem_b)
        ga.wait()
        pltpu.sync_copy(msg_a, agg_sh.at[dst_a], add=True)
        gb.wait()
        pltpu.sync_copy(msg_b, agg_sh.at[dst_b], add=True)

    plsc.subcore_barrier()

    cb = pl.multiple_of(wid * NPAD, 128)
    pltpu.sync_copy(cnt_v, cnt_hbm.at[pl.ds(cb, NPAD)])
    gcb = pl.multiple_of(wid * GPAD, 128)
    pltpu.sync_copy(gcv_v, gc_hbm.at[pl.ds(gcb, GPAD)])
    pltpu.sync_copy(agg_sh.at[pl.ds(rb, 640)], p_hbm.at[c, pl.ds(rb, 640)])


def _tc_body(p, cnt, h, wlT, wrT, bl, o):
    deg = jnp.sum(cnt[...], axis=0).reshape(_BR, 1)
    deg = jnp.maximum(deg, 1.0)
    mean = (p[0] + p[1]) / deg
    acc = jnp.dot(mean, wlT[...], preferred_element_type=_f32)
    acc = acc + jnp.dot(h[...], wrT[...], preferred_element_type=_f32)
    o[...] = jnp.maximum(acc + bl[0:1, :], 0.0)


_BR = 512

_tc_call = pl.pallas_call(
    _tc_body,
    grid=(NPAD // _BR,),
    in_specs=[
        pl.BlockSpec((2, _BR, D), lambda i: (0, i, 0)),
        pl.BlockSpec((NW, _BR), lambda i: (0, i)),
        pl.BlockSpec((_BR, D), lambda i: (i, 0)),
        pl.BlockSpec((D, D), lambda i: (0, 0)),
        pl.BlockSpec((D, D), lambda i: (0, 0)),
        pl.BlockSpec((8, D), lambda i: (0, 0)),
    ],
    out_specs=pl.BlockSpec((_BR, D), lambda i: (i, 0)),
    out_shape=jax.ShapeDtypeStruct((NPAD, D), _f32),
)


@functools.partial(
    pl.kernel,
    out_type=jax.ShapeDtypeStruct((G,), _f32),
    mesh=_mesh,
    compiler_params=_sc_params,
    scratch_types=[
        pltpu.VMEM((NW * GPAD,), _f32),
        pltpu.VMEM((GPAD + 16,), _f32),
        pltpu.VMEM((NPAD,), _i32),
        pltpu.VMEM((64, D), _f32),
        pltpu.VMEM((16, D), _f32),
        pltpu.VMEM((16, D), _f32),
        pltpu.VMEM((2 * D,), _f32),
        pltpu.VMEM((16,), _f32),
        pltpu.VMEM((16,), _f32),
        pltpu.SemaphoreType.DMA,
    ],
)
def _pool_kernel(out_hbm, batch_hbm, gc_all_hbm, wout_hbm, bout_hbm,
                 res_hbm, gc_v, gct_v, batch_v, rowbuf, accmax, accsum,
                 wout_v, bout_v, res_v, sem):
    c = lax.axis_index("c")
    s = lax.axis_index("s")
    wid = c * 16 + s
    wid16 = wid * 16

    pltpu.sync_copy(gc_all_hbm, gc_v)

    @pl.loop(0, GPAD, step=16)
    def _(r):
        t = jnp.zeros((16,), _f32)
        for w in range(NW):
            t = t + gc_v[pl.ds(w * GPAD + r, 16)]
        gct_v[pl.ds(r, 16)] = t

    gct_v[pl.ds(GPAD, 16)] = jnp.zeros((16,), _f32)
    pltpu.sync_copy(batch_hbm, batch_v)
    pltpu.sync_copy(wout_hbm, wout_v)
    pltpu.sync_copy(bout_hbm, bout_v)

    @pl.loop(0, 16)
    def _(r):
        @pl.loop(0, D, step=16)
        def _(cc):
            accmax[r, pl.ds(cc, 16)] = jnp.zeros((16,), _f32)
            accsum[r, pl.ds(cc, 16)] = jnp.zeros((16,), _f32)

    def _cnt_at(g):
        return gct_v[pl.ds(g, 16)][0]

    start_f = lax.fori_loop(0, wid16, lambda g, a: a + _cnt_at(g), 0.0)
    cgs = [_cnt_at(wid16 + gl) for gl in range(16)]
    total_f = cgs[0]
    for gl in range(1, 16):
        total_f = total_f + cgs[gl]
    start = start_f.astype(_i32)
    total = total_f.astype(_i32)

    end = start + total
    w0 = start // 64

    def _chunk(i, carry):
        p = pl.multiple_of((w0 + i) * 64, 64)
        pltpu.sync_copy(out_hbm.at[pl.ds(p, 64)], rowbuf)
        lo = jnp.maximum(start - p, 0)
        hi = jnp.minimum(end - p, 64)

        @pl.loop(0, 64)
        def _(r):
            @pl.when((r >= lo) & (r < hi))
            def _():
                gloc = batch_v[pl.ds(p + r, 16)][0] - wid16

                @pl.loop(0, D, step=16)
                def _(cc):
                    v = rowbuf[r, pl.ds(cc, 16)]
                    accmax[gloc, pl.ds(cc, 16)] = jnp.maximum(
                        accmax[gloc, pl.ds(cc, 16)], v)
                    accsum[gloc, pl.ds(cc, 16)] = accsum[gloc, pl.ds(cc, 16)] + v

        return carry

    nch = jnp.where(total > 0, (end + 63) // 64 - w0, 0)
    lax.fori_loop(0, nch, _chunk, 0)

    lane = lax.iota(_i32, 16)
    bias = bout_v[...][0]
    logits = jnp.zeros((16,), _f32)
    for gl in range(16):
        denom = jnp.maximum(cgs[gl], 1.0)
        acc = jnp.zeros((16,), _f32)
        for cc in range(8):
            acc = acc + accmax[gl, pl.ds(cc * 16, 16)] * wout_v[pl.ds(cc * 16, 16)]
            acc = acc + accsum[gl, pl.ds(cc * 16, 16)] * (
                wout_v[pl.ds(D + cc * 16, 16)] / denom)
        t = jnp.sum(acc) + bias
        logits = jnp.where(lane == gl, t, logits)

    res_v[...] = 1.0 / (1.0 + jnp.exp(-logits))
    pltpu.sync_copy(res_v, res_hbm.at[pl.ds(wid16, 16)])


def kernel(x, edge_index, batch, emb_table, W_l, b_l, W_r, W_out, b_out):
    xp = jnp.concatenate([x[:, 0], jnp.zeros((NPAD - N,), _i32)])
    srcp = jnp.concatenate([edge_index[0], jnp.zeros((EPAD - E,), _i32)])
    dstp = jnp.concatenate([edge_index[1], jnp.full((EPAD - E,), TRASH, _i32)])
    sdp = jnp.stack([srcp.reshape(-1, 64), dstp.reshape(-1, 64)],
                    axis=1).reshape(-1)
    batchp = jnp.concatenate([batch, jnp.full((NPAD - N,), G, _i32)])
    wlT = W_l.T
    wrT = W_r.T
    bl8 = jnp.broadcast_to(b_l[None, :], (8, D))
    woutv = W_out.reshape(-1)
    boutv = jnp.pad(b_out, (0, 15))

    h, p, cnt, gc = _agg_kernel(emb_table, xp, sdp, batchp)
    cnt2 = cnt.reshape(NW, NPAD)
    out = _tc_call(p, cnt2, h, wlT, wrT, bl8)
    res = _pool_kernel(out, batchp, gc.reshape(-1), woutv, boutv)
    return res
